# Initial kernel scaffold; baseline (speedup 1.0000x reference)
import jax
import jax.numpy as jnp
from jax.experimental import pallas as pl


def kernel(sequence, segment_label, token_table, pos_table, seg_table, gamma, beta):
    raise NotImplementedError("write your pallas kernel here")


# speedup vs baseline: 4.3329x; 4.3329x over previous
import functools

import jax
import jax.numpy as jnp
from jax.experimental import pallas as pl
from jax.experimental.pallas import tpu as pltpu
from jax.experimental.pallas import tpu_sc as plsc

PAD = 0
EPS = 1e-5
GW = 128
RB = 1600


def _sc_gather(table, idx_flat, n, d):
    mesh = plsc.VectorSubcoreMesh(core_axis_name="core", subcore_axis_name="subcore")

    @functools.partial(
        pl.kernel,
        out_type=jax.ShapeDtypeStruct((n, d), jnp.float32),
        mesh=mesh,
    )
    def k(table_hbm, i_hbm, o_hbm):
        def body(i_vmem, o_vmem):
            pltpu.sync_copy(table_hbm.at[i_vmem.at[0]], o_vmem)

        pltpu.emit_pipeline(
            body,
            grid=(n // GW,),
            in_specs=[pl.BlockSpec((1, GW), index_map=lambda i: (0, i))],
            out_specs=[pl.BlockSpec((GW, d), index_map=lambda i: (i, 0))],
            core_axis_name=("core", "subcore"),
            dimension_semantics=(pltpu.PARALLEL,),
        )(i_hbm, o_hbm)

    return k(table, idx_flat)


def _tc_body(tok_ref, mask_ref, lab_ref, pos_ref, s0_ref, ds_ref, g_ref, b_ref, o_ref):
    tok = tok_ref[...]
    mask = mask_ref[...]
    lab = lab_ref[...]
    pos = pos_ref[...]
    s0 = s0_ref[...]
    ds = ds_ref[...]
    x = tok * mask + pos + s0 + lab * ds
    mean = jnp.mean(x, axis=-1, keepdims=True)
    xc = x - mean
    var = jnp.mean(xc * xc, axis=-1, keepdims=True)
    y = xc * jax.lax.rsqrt(var + EPS)
    o_ref[...] = y * g_ref[...] + b_ref[...]


def _tc_ln(tok2, maskf, labf, pos_tiled, s0, ds, g2, b2):
    n, d = tok2.shape
    col = lambda i: (i, 0)
    cst = lambda i: (0, 0)
    return pl.pallas_call(
        _tc_body,
        grid=(n // RB,),
        in_specs=[
            pl.BlockSpec((RB, d), col),
            pl.BlockSpec((RB, 1), col),
            pl.BlockSpec((RB, 1), col),
            pl.BlockSpec((RB, d), cst),
            pl.BlockSpec((1, d), cst),
            pl.BlockSpec((1, d), cst),
            pl.BlockSpec((1, d), cst),
            pl.BlockSpec((1, d), cst),
        ],
        out_specs=pl.BlockSpec((RB, d), col),
        out_shape=jax.ShapeDtypeStruct((n, d), jnp.float32),
    )(tok2, maskf, labf, pos_tiled, s0, ds, g2, b2)


def kernel(sequence, segment_label, token_table, pos_table, seg_table, gamma, beta):
    b, s = sequence.shape
    v, d = token_table.shape
    n = b * s
    seq_i = sequence.astype(jnp.int32).reshape(n)
    tok2 = _sc_gather(token_table, seq_i.reshape(1, n), n, d)
    maskf = (seq_i != PAD).astype(jnp.float32).reshape(n, 1)
    labf = segment_label.astype(jnp.float32).reshape(n, 1)
    pos_tiled = jnp.tile(pos_table[:s], (RB // s, 1))
    s0 = seg_table[0:1]
    ds = seg_table[1:2] - seg_table[0:1]
    out2 = _tc_ln(tok2, maskf, labf, pos_tiled, s0, ds, gamma[None], beta[None])
    return out2.reshape(b, s, d)


# speedup vs baseline: 5.1390x; 1.1860x over previous
import functools

import jax
import jax.numpy as jnp
from jax.experimental import pallas as pl
from jax.experimental.pallas import tpu as pltpu
from jax.experimental.pallas import tpu_sc as plsc

PAD = 0
EPS = 1e-5
GW = 128
RB = 1600


def _sc_gather(table, idx_flat, n, d):
    mesh = plsc.VectorSubcoreMesh(core_axis_name="core", subcore_axis_name="subcore")

    @functools.partial(
        pl.kernel,
        out_type=jax.ShapeDtypeStruct((n, d), jnp.float32),
        mesh=mesh,
    )
    def k(table_hbm, i_hbm, o_hbm):
        def body(i_vmem, o_vmem):
            pltpu.sync_copy(table_hbm.at[i_vmem.at[0]], o_vmem)

        pltpu.emit_pipeline(
            body,
            grid=(n // GW,),
            in_specs=[pl.BlockSpec((1, GW), index_map=lambda i: (0, i))],
            out_specs=[pl.BlockSpec((GW, d), index_map=lambda i: (i, 0))],
            core_axis_name=("core", "subcore"),
            dimension_semantics=(pltpu.PARALLEL,),
        )(i_hbm, o_hbm)

    return k(table, idx_flat)


def _tc_body(tok_ref, col_ref, pos_ref, ds_ref, row0_ref, g_ref, b_ref, o_ref):
    tok = tok_ref[...]
    col = col_ref[...]
    pos = pos_ref[...]
    ds = ds_ref[...]
    row0 = row0_ref[...]
    padf = jnp.floor(col * 0.5)
    labf = col - 2.0 * padf
    x = tok + pos + labf * ds - padf * row0
    mean = jnp.mean(x, axis=-1, keepdims=True)
    xc = x - mean
    var = jnp.mean(xc * xc, axis=-1, keepdims=True)
    y = xc * jax.lax.rsqrt(var + EPS)
    y = y * g_ref[...] + b_ref[...]
    o_ref[...] = y.reshape(o_ref.shape)


def _tc_ln(tok2, colf, pos_tiled, ds, row0, g2, b2, b, s, d):
    n = b * s
    bb = RB // s
    col = lambda i: (i, 0)
    cst = lambda i: (0, 0)
    return pl.pallas_call(
        _tc_body,
        grid=(n // RB,),
        in_specs=[
            pl.BlockSpec((RB, d), col),
            pl.BlockSpec((RB, 1), col),
            pl.BlockSpec((RB, d), cst),
            pl.BlockSpec((1, d), cst),
            pl.BlockSpec((1, d), cst),
            pl.BlockSpec((1, d), cst),
            pl.BlockSpec((1, d), cst),
        ],
        out_specs=pl.BlockSpec((bb, s, d), lambda i: (i, 0, 0)),
        out_shape=jax.ShapeDtypeStruct((b, s, d), jnp.float32),
        compiler_params=pltpu.CompilerParams(
            dimension_semantics=("parallel",),
        ),
    )(tok2, colf, pos_tiled, ds, row0, g2, b2)


def kernel(sequence, segment_label, token_table, pos_table, seg_table, gamma, beta):
    b, s = sequence.shape
    v, d = token_table.shape
    n = b * s
    seq_i = sequence.astype(jnp.int32).reshape(n)
    tok2 = _sc_gather(token_table, seq_i.reshape(1, n), n, d)
    code = segment_label.astype(jnp.int32).reshape(n) + 2 * (seq_i == PAD)
    colf = code.astype(jnp.float32).reshape(n, 1)
    pos_tiled = jnp.tile(pos_table[:s] + seg_table[0:1], (RB // s, 1))
    ds = seg_table[1:2] - seg_table[0:1]
    row0 = token_table[0:1]
    return _tc_ln(tok2, colf, pos_tiled, ds, row0, gamma[None], beta[None], b, s, d)


# speedup vs baseline: 6.6279x; 1.2897x over previous
import functools

import jax
import jax.numpy as jnp
from jax.experimental import pallas as pl
from jax.experimental.pallas import tpu as pltpu
from jax.experimental.pallas import tpu_sc as plsc

PAD = 0
EPS = 1e-5
GW = 128
RB = 3200
GU = RB // 128


def _sc_gather(table, idx_flat, n, d):
    mesh = plsc.VectorSubcoreMesh(core_axis_name="core", subcore_axis_name="subcore")

    @functools.partial(
        pl.kernel,
        out_type=jax.ShapeDtypeStruct((n, d), jnp.float32),
        mesh=mesh,
    )
    def k(table_hbm, i_hbm, o_hbm):
        def body(i_vmem, o_vmem):
            pltpu.sync_copy(table_hbm.at[i_vmem.at[0]], o_vmem)

        pltpu.emit_pipeline(
            body,
            grid=(n // GW,),
            in_specs=[pl.BlockSpec((1, GW), index_map=lambda i: (0, i))],
            out_specs=[pl.BlockSpec((GW, d), index_map=lambda i: (i, 0))],
            core_axis_name=("core", "subcore"),
            dimension_semantics=(pltpu.PARALLEL,),
        )(i_hbm, o_hbm)

    return k(table, idx_flat)


def _tc_body(tok_ref, g_ref, pos_ref, n2_ref, gb_ref, o_ref):
    tok = tok_ref[...]
    gcode = g_ref[0]
    pos = pos_ref[...]
    n2 = n2_ref[...]
    padg = jnp.floor(gcode * 0.5)
    labg = gcode - 2.0 * padg
    pieces = []
    for u in range(GU):
        m = jnp.concatenate([labg[u:u + 1], padg[u:u + 1]], axis=0)
        pieces.append(jax.lax.dot_general(
            m, n2, (((0,), (0,)), ((), ())),
            precision=jax.lax.Precision.HIGHEST))
    x = tok + pos + jnp.concatenate(pieces, axis=0)
    mean = jnp.mean(x, axis=-1, keepdims=True)
    xc = x - mean
    var = jnp.mean(xc * xc, axis=-1, keepdims=True)
    y = xc * jax.lax.rsqrt(var + EPS)
    y = y * gb_ref[0:1] + gb_ref[1:2]
    o_ref[...] = y.reshape(o_ref.shape)


def _tc_ln(tok2, gcode, pos_tiled, n2, gb, b, s, d):
    n = b * s
    bb = RB // s
    col = lambda i: (i, 0)
    cst = lambda i: (0, 0)
    return pl.pallas_call(
        _tc_body,
        grid=(n // RB,),
        in_specs=[
            pl.BlockSpec((RB, d), col),
            pl.BlockSpec((1, GU, 128), lambda i: (i, 0, 0)),
            pl.BlockSpec((RB, d), cst),
            pl.BlockSpec((2, d), cst),
            pl.BlockSpec((2, d), cst),
        ],
        out_specs=pl.BlockSpec((bb, s, d), lambda i: (i, 0, 0)),
        out_shape=jax.ShapeDtypeStruct((b, s, d), jnp.float32),
        compiler_params=pltpu.CompilerParams(
            dimension_semantics=("parallel",),
        ),
    )(tok2, gcode, pos_tiled, n2, gb)


def kernel(sequence, segment_label, token_table, pos_table, seg_table, gamma, beta):
    b, s = sequence.shape
    v, d = token_table.shape
    n = b * s
    seq_i = sequence.astype(jnp.int32).reshape(n)
    tok2 = _sc_gather(token_table, seq_i.reshape(1, n), n, d)
    code = segment_label.astype(jnp.int32) + 2 * (sequence.astype(jnp.int32) == PAD)
    gcode = code.astype(jnp.float32).reshape(n // RB, GU, 128)
    pos_tiled = jnp.tile(pos_table[:s] + seg_table[0:1], (RB // s, 1))
    n2 = jnp.concatenate([seg_table[1:2] - seg_table[0:1], -token_table[0:1]], axis=0)
    gb = jnp.concatenate([gamma[None], beta[None]], axis=0)
    return _tc_ln(tok2, gcode, pos_tiled, n2, gb, b, s, d)


# speedup vs baseline: 7.8840x; 1.1895x over previous
import functools

import jax
import jax.numpy as jnp
from jax.experimental import pallas as pl
from jax.experimental.pallas import tpu as pltpu
from jax.experimental.pallas import tpu_sc as plsc

PAD = 0
EPS = 1e-5
GW = 128
RB = 3200
GU = RB // 128
NCHUNK = 4


def _sc_gather(table, idx_flat, n, d):
    mesh = plsc.VectorSubcoreMesh(core_axis_name="core", subcore_axis_name="subcore")

    @functools.partial(
        pl.kernel,
        out_type=jax.ShapeDtypeStruct((n, d), jnp.float32),
        mesh=mesh,
    )
    def k(table_hbm, i_hbm, o_hbm):
        def body(i_vmem, o_vmem):
            pltpu.sync_copy(table_hbm.at[i_vmem.at[0]], o_vmem)

        pltpu.emit_pipeline(
            body,
            grid=(n // GW,),
            in_specs=[pl.BlockSpec((1, GW), index_map=lambda i: (0, i))],
            out_specs=[pl.BlockSpec((GW, d), index_map=lambda i: (i, 0))],
            core_axis_name=("core", "subcore"),
            dimension_semantics=(pltpu.PARALLEL,),
        )(i_hbm, o_hbm)

    return k(table, idx_flat)


def _tc_body(tok_ref, g_ref, pos_ref, n2_ref, gb_ref, o_ref):
    tok = tok_ref[...]
    gcode = g_ref[0]
    pos = pos_ref[...]
    n2 = n2_ref[...]
    padg = jnp.floor(gcode * 0.5)
    labg = gcode - 2.0 * padg
    pieces = []
    for u in range(GU):
        m = jnp.concatenate([labg[u:u + 1], padg[u:u + 1]], axis=0)
        pieces.append(jax.lax.dot_general(
            m, n2, (((0,), (0,)), ((), ())),
            precision=jax.lax.Precision.HIGHEST))
    x = tok + pos + jnp.concatenate(pieces, axis=0)
    mean = jnp.mean(x, axis=-1, keepdims=True)
    xc = x - mean
    var = jnp.mean(xc * xc, axis=-1, keepdims=True)
    y = xc * jax.lax.rsqrt(var + EPS)
    y = y * gb_ref[0:1] + gb_ref[1:2]
    o_ref[...] = y.reshape(o_ref.shape)


def _tc_body_alias(buf_ref, tok_ref, g_ref, pos_ref, n2_ref, gb_ref, o_ref):
    _tc_body(tok_ref, g_ref, pos_ref, n2_ref, gb_ref, o_ref)


def _tc_ln_chunk(chunk, prev_buf, tok_c, gcode, pos_tiled, n2, gb, b, s, d):
    n = b * s
    nc = n // NCHUNK
    nblk = nc // RB
    bb = RB // s
    base = chunk * nblk
    col = lambda i: (i, 0)
    cst = lambda i: (0, 0)
    in_specs = [
        pl.BlockSpec((RB, d), col),
        pl.BlockSpec((1, GU, 128), lambda i: (i + base, 0, 0)),
        pl.BlockSpec((RB, d), cst),
        pl.BlockSpec((2, d), cst),
        pl.BlockSpec((2, d), cst),
    ]
    out_spec = pl.BlockSpec((bb, s, d), lambda i: (i + base, 0, 0))
    out_shape = jax.ShapeDtypeStruct((b, s, d), jnp.float32)
    params = pltpu.CompilerParams(dimension_semantics=("parallel",))
    if prev_buf is None:
        return pl.pallas_call(
            _tc_body,
            grid=(nblk,),
            in_specs=in_specs,
            out_specs=out_spec,
            out_shape=out_shape,
            compiler_params=params,
        )(tok_c, gcode, pos_tiled, n2, gb)
    return pl.pallas_call(
        _tc_body_alias,
        grid=(nblk,),
        in_specs=[pl.BlockSpec(memory_space=pl.ANY)] + in_specs,
        out_specs=out_spec,
        out_shape=out_shape,
        input_output_aliases={0: 0},
        compiler_params=params,
    )(prev_buf, tok_c, gcode, pos_tiled, n2, gb)


def kernel(sequence, segment_label, token_table, pos_table, seg_table, gamma, beta):
    b, s = sequence.shape
    v, d = token_table.shape
    n = b * s
    nc = n // NCHUNK
    seq_i = sequence.astype(jnp.int32).reshape(1, n)
    code = segment_label.astype(jnp.int32) + 2 * (sequence.astype(jnp.int32) == PAD)
    gcode = code.astype(jnp.float32).reshape(n // RB, GU, 128)
    pos_tiled = jnp.tile(pos_table[:s] + seg_table[0:1], (RB // s, 1))
    n2 = jnp.concatenate([seg_table[1:2] - seg_table[0:1], -token_table[0:1]], axis=0)
    gb = jnp.concatenate([gamma[None], beta[None]], axis=0)

    toks = [
        _sc_gather(token_table, jax.lax.slice(seq_i, (0, k * nc), (1, (k + 1) * nc)), nc, d)
        for k in range(NCHUNK)
    ]
    buf = None
    for k in range(NCHUNK):
        buf = _tc_ln_chunk(k, buf, toks[k], gcode, pos_tiled, n2, gb, b, s, d)
    return buf
